# Initial kernel scaffold; baseline (speedup 1.0000x reference)
import jax
import jax.numpy as jnp
from jax.experimental import pallas as pl


def kernel(x, c, adj_t, edge_w, Wl1, Wr1, att1, b1, Wl2, Wr2, att2, b2, Wfc, bfc):
    raise NotImplementedError("write your pallas kernel here")


# speedup vs baseline: 24.7829x; 24.7829x over previous
import functools

import jax
import jax.numpy as jnp
from jax import lax
from jax.experimental import pallas as pl
from jax.experimental.pallas import tpu as pltpu
from jax.experimental.pallas import tpu_sc as plsc

N = 10000
E = 320000
D_IN = 128
D_H = 32

NC = 2
NS = 16
NW = NC * NS

N_PAD = 10240
ROWS_PER_TILE = N_PAD // NS

B = 96
E_TOT = E + N
E_W = 10368
E_PAD = E_W * NW
CHUNKS = E_W // B

ACC_W = 64

_f32 = jnp.float32


_BLK = 1024
_GRID = N_PAD // _BLK


def _mm_body(x_ref, w_ref, o_ref):
    o_ref[...] = jnp.dot(x_ref[...], w_ref[...], preferred_element_type=_f32)


def _tc_matmul(x, w):
    m, k = x.shape
    n = w.shape[1]
    return pl.pallas_call(
        _mm_body,
        grid=(m // _BLK,),
        in_specs=[
            pl.BlockSpec((_BLK, k), lambda i: (i, 0)),
            pl.BlockSpec((k, n), lambda i: (0, 0)),
        ],
        out_specs=pl.BlockSpec((_BLK, n), lambda i: (i, 0)),
        out_shape=jax.ShapeDtypeStruct((m, n), _f32),
    )(x, w)


def _combine_body(acc_ref, b_ref, w_ref, bw_ref, o_ref):
    nm = acc_ref[0, :, :D_H] + acc_ref[1, :, :D_H]
    dn = jnp.maximum(acc_ref[0, :, D_H] + acc_ref[1, :, D_H], 1e-37)
    h = jnp.maximum(nm / dn[:, None] + b_ref[0], 0.0)
    o_ref[...] = jnp.dot(h, w_ref[...], preferred_element_type=_f32) + bw_ref[0]


def _tc_combine(acc, b, w, bw):
    n_out = w.shape[1]
    return pl.pallas_call(
        _combine_body,
        grid=(_GRID,),
        in_specs=[
            pl.BlockSpec((NC, _BLK, ACC_W), lambda i: (0, i, 0)),
            pl.BlockSpec((1, D_H), lambda i: (0, 0)),
            pl.BlockSpec((D_H, n_out), lambda i: (0, 0)),
            pl.BlockSpec((1, n_out), lambda i: (0, 0)),
        ],
        out_specs=pl.BlockSpec((_BLK, n_out), lambda i: (i, 0)),
        out_shape=jax.ShapeDtypeStruct((N_PAD, n_out), _f32),
    )(acc, b, w, bw)


def _sc_body(xl_hbm, xr_hbm, att_hbm, src_hbm, dst_hbm,
             acc_hbm,
             xl_sh, xr_sh, acc_sh,
             src_v, dst_v, xls_v, xrd_v, upd_v, att_v,
             sem1, sem2):
    cid = lax.axis_index("c")
    sid = lax.axis_index("s")
    wid = sid * NC + cid
    row0 = sid * ROWS_PER_TILE

    zero16 = jnp.zeros((16,), _f32)

    def _z(i, _):
        upd_v[i, pl.ds(0, 16)] = zero16
        upd_v[i, pl.ds(16, 16)] = zero16
        upd_v[i, pl.ds(32, 16)] = zero16
        upd_v[i, pl.ds(48, 16)] = zero16
        return 0
    lax.fori_loop(0, B, _z, 0)

    def _zcp(i, _):
        pltpu.sync_copy(upd_v, acc_sh.at[pl.ds(row0 + i * B, B)])
        return 0
    lax.fori_loop(0, ROWS_PER_TILE // B, _zcp, 0)
    rem = ROWS_PER_TILE - (ROWS_PER_TILE // B) * B
    if rem:
        pltpu.sync_copy(upd_v.at[pl.ds(0, rem)],
                        acc_sh.at[pl.ds(row0 + (ROWS_PER_TILE // B) * B, rem)])

    pltpu.sync_copy(xl_hbm.at[pl.ds(row0, ROWS_PER_TILE)],
                    xl_sh.at[pl.ds(row0, ROWS_PER_TILE)])
    pltpu.sync_copy(xr_hbm.at[pl.ds(row0, ROWS_PER_TILE)],
                    xr_sh.at[pl.ds(row0, ROWS_PER_TILE)])

    pltpu.sync_copy(att_hbm, att_v)
    att0 = att_v[0, pl.ds(0, 16)]
    att1 = att_v[0, pl.ds(16, 16)]

    plsc.subcore_barrier()

    iota16 = lax.broadcasted_iota(jnp.int32, (16,), 0)

    def _chunk(k, _):
        base = pl.multiple_of(wid * E_W + k * B, 8)
        pltpu.sync_copy(src_hbm.at[pl.ds(base, B)], src_v)
        pltpu.sync_copy(dst_hbm.at[pl.ds(base, B)], dst_v)
        cp1 = pltpu.async_copy(xl_sh.at[src_v], xls_v, sem1)
        cp2 = pltpu.async_copy(xr_sh.at[dst_v], xrd_v, sem2)
        cp1.wait()
        cp2.wait()

        def _group(g, _):
            e0 = g * 16
            sums = jnp.zeros((16,), _f32)
            for i in range(16):
                e = e0 + i
                a0 = xls_v[e, pl.ds(0, 16)]
                a1 = xls_v[e, pl.ds(16, 16)]
                s0 = a0 + xrd_v[e, pl.ds(0, 16)]
                s1 = a1 + xrd_v[e, pl.ds(16, 16)]
                l0 = jnp.maximum(s0, 0.2 * s0)
                l1 = jnp.maximum(s1, 0.2 * s1)
                p = l0 * att0 + l1 * att1
                si = jnp.sum(p)
                sums = jnp.where(iota16 == i, si, sums)
            ex = jnp.exp(sums)
            for i in range(16):
                e = e0 + i
                exi = ex[i]
                upd_v[e, pl.ds(0, 16)] = exi * xls_v[e, pl.ds(0, 16)]
                upd_v[e, pl.ds(16, 16)] = exi * xls_v[e, pl.ds(16, 16)]
                upd_v[e, pl.ds(32, 16)] = jnp.where(iota16 == 0, exi, 0.0)
            return 0
        lax.fori_loop(0, B // 16, _group, 0)

        pltpu.sync_copy(upd_v, acc_sh.at[dst_v], add=True)
        return 0

    lax.fori_loop(0, CHUNKS, _chunk, 0)

    plsc.subcore_barrier()

    pltpu.sync_copy(acc_sh.at[pl.ds(row0, ROWS_PER_TILE)],
                    acc_hbm.at[cid, pl.ds(row0, ROWS_PER_TILE)])


@functools.cache
def _make_sc_attention():
  return pl.kernel(
    _sc_body,
    out_type=jax.ShapeDtypeStruct((NC, N_PAD, ACC_W), _f32),
    mesh=plsc.VectorSubcoreMesh(core_axis_name="c", subcore_axis_name="s",
                                num_cores=NC, num_subcores=NS),
    compiler_params=pltpu.CompilerParams(needs_layout_passes=False),
    scratch_types=[
        pltpu.VMEM_SHARED((N_PAD, D_H), _f32),
        pltpu.VMEM_SHARED((N_PAD, D_H), _f32),
        pltpu.VMEM_SHARED((N_PAD, ACC_W), _f32),
        pltpu.VMEM((B,), jnp.int32),
        pltpu.VMEM((B,), jnp.int32),
        pltpu.VMEM((B, D_H), _f32),
        pltpu.VMEM((B, D_H), _f32),
        pltpu.VMEM((B, ACC_W), _f32),
        pltpu.VMEM((1, D_H), _f32),
        pltpu.SemaphoreType.DMA,
        pltpu.SemaphoreType.DMA,
    ],
  )


def kernel(x, c, adj_t, edge_w, Wl1, Wr1, att1, b1, Wl2, Wr2, att2, b2, Wfc, bfc):
    del c, edge_w
    loop = jnp.arange(N, dtype=jnp.int32)
    npad = E_PAD - E_TOT
    src = jnp.concatenate([adj_t[0], loop,
                           jnp.zeros((npad,), jnp.int32)])
    dst = jnp.concatenate([adj_t[1], loop,
                           jnp.full((npad,), N, jnp.int32)])

    x_pad = jnp.pad(x, ((0, N_PAD - N), (0, 0)))

    xlr1 = _tc_matmul(x_pad, jnp.concatenate([Wl1, Wr1], axis=1))
    xl1 = xlr1[:, :D_H]
    xr1 = xlr1[:, D_H:]
    sc_attention = _make_sc_attention()
    acc1 = sc_attention(xl1, xr1, att1.reshape(1, D_H), src, dst)

    xlr2 = _tc_combine(acc1, b1.reshape(1, D_H),
                       jnp.concatenate([Wl2, Wr2], axis=1),
                       jnp.zeros((1, 2 * D_H), _f32))
    xl2 = xlr2[:, :D_H]
    xr2 = xlr2[:, D_H:]
    acc2 = sc_attention(xl2, xr2, att2.reshape(1, D_H), src, dst)

    out = _tc_combine(acc2, b2.reshape(1, D_H), Wfc,
                      bfc.reshape(1, 1))
    return out[:N]


# speedup vs baseline: 31.5233x; 1.2720x over previous
import functools

import jax
import jax.numpy as jnp
from jax import lax
from jax.experimental import pallas as pl
from jax.experimental.pallas import tpu as pltpu
from jax.experimental.pallas import tpu_sc as plsc

N = 10000
E = 320000
D_IN = 128
D_H = 32

NC = 2
NS = 16
NW = NC * NS

N_PAD = 10240
ROWS_PER_TILE = N_PAD // NS

B = 64
E_TOT = E + N
E_W = 10368
E_PAD = E_W * NW
CHUNKS = E_W // B
PAIRS = CHUNKS // 2

ACC_W = 64

_f32 = jnp.float32


_BLK = 1024
_GRID = N_PAD // _BLK


def _mm_body(x_ref, w_ref, o_ref):
    o_ref[...] = jnp.dot(x_ref[...], w_ref[...], preferred_element_type=_f32)


def _tc_matmul(x, w):
    m, k = x.shape
    n = w.shape[1]
    return pl.pallas_call(
        _mm_body,
        grid=(m // _BLK,),
        in_specs=[
            pl.BlockSpec((_BLK, k), lambda i: (i, 0)),
            pl.BlockSpec((k, n), lambda i: (0, 0)),
        ],
        out_specs=pl.BlockSpec((_BLK, n), lambda i: (i, 0)),
        out_shape=jax.ShapeDtypeStruct((m, n), _f32),
    )(x, w)


def _combine_body(acc_ref, b_ref, w_ref, bw_ref, o_ref):
    nm = acc_ref[0, :, :D_H] + acc_ref[1, :, :D_H]
    dn = jnp.maximum(acc_ref[0, :, D_H] + acc_ref[1, :, D_H], 1e-37)
    h = jnp.maximum(nm / dn[:, None] + b_ref[0], 0.0)
    o_ref[...] = jnp.dot(h, w_ref[...], preferred_element_type=_f32) + bw_ref[0]


def _tc_combine(acc, b, w, bw):
    n_out = w.shape[1]
    return pl.pallas_call(
        _combine_body,
        grid=(_GRID,),
        in_specs=[
            pl.BlockSpec((NC, _BLK, ACC_W), lambda i: (0, i, 0)),
            pl.BlockSpec((1, D_H), lambda i: (0, 0)),
            pl.BlockSpec((D_H, n_out), lambda i: (0, 0)),
            pl.BlockSpec((1, n_out), lambda i: (0, 0)),
        ],
        out_specs=pl.BlockSpec((_BLK, n_out), lambda i: (i, 0)),
        out_shape=jax.ShapeDtypeStruct((N_PAD, n_out), _f32),
    )(acc, b, w, bw)


def _sc_body(xl_hbm, xr_hbm, att_hbm, src_hbm, dst_hbm,
             acc_hbm,
             xl_sh, xr_sh, acc_sh,
             idx_v,
             xls0_v, xrd0_v, xls1_v, xrd1_v, upd0_v, att_v,
             i0a, i0b, i1a, i1b, g0a, g0b, g1a, g1b, s0, s1):
    cid = lax.axis_index("c")
    sid = lax.axis_index("s")
    wid = sid * NC + cid
    row0 = sid * ROWS_PER_TILE

    zero16 = jnp.zeros((16,), _f32)

    def _z(i, _):
        upd0_v[i, pl.ds(0, 16)] = zero16
        upd0_v[i, pl.ds(16, 16)] = zero16
        upd0_v[i, pl.ds(32, 16)] = zero16
        upd0_v[i, pl.ds(48, 16)] = zero16
        return 0
    lax.fori_loop(0, B, _z, 0)

    def _zcp(i, _):
        pltpu.sync_copy(upd0_v.at[pl.ds(0, 64)],
                        acc_sh.at[pl.ds(row0 + i * 64, 64)])
        return 0
    lax.fori_loop(0, ROWS_PER_TILE // 64, _zcp, 0)

    pltpu.sync_copy(xl_hbm.at[pl.ds(row0, ROWS_PER_TILE)],
                    xl_sh.at[pl.ds(row0, ROWS_PER_TILE)])
    pltpu.sync_copy(xr_hbm.at[pl.ds(row0, ROWS_PER_TILE)],
                    xr_sh.at[pl.ds(row0, ROWS_PER_TILE)])

    pltpu.sync_copy(att_hbm, att_v)
    att0 = att_v[0, pl.ds(0, 16)]
    att1 = att_v[0, pl.ds(16, 16)]

    plsc.subcore_barrier()

    iota16 = lax.broadcasted_iota(jnp.int32, (16,), 0)
    ebase = wid * E_W

    def _compute(xls_v, xrd_v, upd_v):
        def _group(g, _):
            e0 = g * 16
            sums = jnp.zeros((16,), _f32)
            for i in range(16):
                e = e0 + i
                a0 = xls_v[e, pl.ds(0, 16)]
                a1 = xls_v[e, pl.ds(16, 16)]
                s0_ = a0 + xrd_v[e, pl.ds(0, 16)]
                s1_ = a1 + xrd_v[e, pl.ds(16, 16)]
                l0 = jnp.maximum(s0_, 0.2 * s0_)
                l1 = jnp.maximum(s1_, 0.2 * s1_)
                p = l0 * att0 + l1 * att1
                si = jnp.sum(p)
                sums = jnp.where(iota16 == i, si, sums)
            ex = jnp.exp(sums)
            for i in range(16):
                e = e0 + i
                exi = ex[i]
                upd_v[e, pl.ds(0, 16)] = exi * xls_v[e, pl.ds(0, 16)]
                upd_v[e, pl.ds(16, 16)] = exi * xls_v[e, pl.ds(16, 16)]
                upd_v[e, pl.ds(32, 16)] = jnp.where(iota16 == 0, exi, 0.0)
            return 0
        lax.fori_loop(0, B // 16, _group, 0)

    def _pair(t, _):
        base = pl.multiple_of(ebase + t * 2 * B, 8)
        base1 = pl.multiple_of(base + B, 8)
        ld0a = pltpu.async_copy(src_hbm.at[pl.ds(base, B)], idx_v.at[0], i0a)
        ld0b = pltpu.async_copy(dst_hbm.at[pl.ds(base, B)], idx_v.at[1], i0b)
        ld1a = pltpu.async_copy(src_hbm.at[pl.ds(base1, B)], idx_v.at[2], i1a)
        ld1b = pltpu.async_copy(dst_hbm.at[pl.ds(base1, B)], idx_v.at[3], i1b)
        ld0a.wait()
        ld0b.wait()
        cp0a = pltpu.async_copy(xl_sh.at[idx_v.at[0]], xls0_v, g0a)
        cp0b = pltpu.async_copy(xr_sh.at[idx_v.at[1]], xrd0_v, g0b)
        ld1a.wait()
        ld1b.wait()
        cp1a = pltpu.async_copy(xl_sh.at[idx_v.at[2]], xls1_v, g1a)
        cp1b = pltpu.async_copy(xr_sh.at[idx_v.at[3]], xrd1_v, g1b)

        cp0a.wait()
        cp0b.wait()
        _compute(xls0_v, xrd0_v, upd0_v)
        pltpu.sync_copy(upd0_v, acc_sh.at[idx_v.at[1]], add=True)

        cp1a.wait()
        cp1b.wait()
        _compute(xls1_v, xrd1_v, upd0_v)
        pltpu.sync_copy(upd0_v, acc_sh.at[idx_v.at[3]], add=True)
        return 0

    lax.fori_loop(0, PAIRS, _pair, 0)

    plsc.subcore_barrier()

    pltpu.sync_copy(acc_sh.at[pl.ds(row0, ROWS_PER_TILE)],
                    acc_hbm.at[cid, pl.ds(row0, ROWS_PER_TILE)])


@functools.cache
def _make_sc_attention():
  return pl.kernel(
    _sc_body,
    out_type=jax.ShapeDtypeStruct((NC, N_PAD, ACC_W), _f32),
    mesh=plsc.VectorSubcoreMesh(core_axis_name="c", subcore_axis_name="s",
                                num_cores=NC, num_subcores=NS),
    compiler_params=pltpu.CompilerParams(needs_layout_passes=False),
    scratch_types=[
        pltpu.VMEM_SHARED((N_PAD, D_H), _f32),
        pltpu.VMEM_SHARED((N_PAD, D_H), _f32),
        pltpu.VMEM_SHARED((N_PAD, ACC_W), _f32),
        pltpu.VMEM((4, B), jnp.int32),
        pltpu.VMEM((B, D_H), _f32),
        pltpu.VMEM((B, D_H), _f32),
        pltpu.VMEM((B, D_H), _f32),
        pltpu.VMEM((B, D_H), _f32),
        pltpu.VMEM((B, ACC_W), _f32),
        pltpu.VMEM((1, D_H), _f32),
        pltpu.SemaphoreType.DMA,
        pltpu.SemaphoreType.DMA,
        pltpu.SemaphoreType.DMA,
        pltpu.SemaphoreType.DMA,
        pltpu.SemaphoreType.DMA,
        pltpu.SemaphoreType.DMA,
        pltpu.SemaphoreType.DMA,
        pltpu.SemaphoreType.DMA,
        pltpu.SemaphoreType.DMA,
        pltpu.SemaphoreType.DMA,
    ],
  )


def kernel(x, c, adj_t, edge_w, Wl1, Wr1, att1, b1, Wl2, Wr2, att2, b2, Wfc, bfc):
    del c, edge_w
    loop = jnp.arange(N, dtype=jnp.int32)
    npad = E_PAD - E_TOT
    src = jnp.concatenate([adj_t[0], loop,
                           jnp.zeros((npad,), jnp.int32)])
    dst = jnp.concatenate([adj_t[1], loop,
                           jnp.full((npad,), N, jnp.int32)])

    x_pad = jnp.pad(x, ((0, N_PAD - N), (0, 0)))

    xlr1 = _tc_matmul(x_pad, jnp.concatenate([Wl1, Wr1], axis=1))
    xl1 = xlr1[:, :D_H]
    xr1 = xlr1[:, D_H:]
    sc_attention = _make_sc_attention()
    acc1 = sc_attention(xl1, xr1, att1.reshape(1, D_H), src, dst)

    xlr2 = _tc_combine(acc1, b1.reshape(1, D_H),
                       jnp.concatenate([Wl2, Wr2], axis=1),
                       jnp.zeros((1, 2 * D_H), _f32))
    xl2 = xlr2[:, :D_H]
    xr2 = xlr2[:, D_H:]
    acc2 = sc_attention(xl2, xr2, att2.reshape(1, D_H), src, dst)

    out = _tc_combine(acc2, b2.reshape(1, D_H), Wfc,
                      bfc.reshape(1, 1))
    return out[:N]


# speedup vs baseline: 33.7978x; 1.0722x over previous
import functools

import jax
import jax.numpy as jnp
from jax import lax
from jax.experimental import pallas as pl
from jax.experimental.pallas import tpu as pltpu
from jax.experimental.pallas import tpu_sc as plsc

N = 10000
E = 320000
D_IN = 128
D_H = 32

NC = 2
NS = 16
NW = NC * NS

N_PAD = 10240
ROWS_PER_TILE = N_PAD // NS

B = 80
E_TOT = E + N
E_W = 10400
E_PAD = E_W * NW
CHUNKS = E_W // B
PAIRS = CHUNKS // 2

ACC_W = 48

_f32 = jnp.float32


_BLK = 1024
_GRID = N_PAD // _BLK


def _mm_body(x_ref, w_ref, o_ref):
    o_ref[...] = jnp.dot(x_ref[...], w_ref[...], preferred_element_type=_f32)


def _tc_matmul(x, w):
    m, k = x.shape
    n = w.shape[1]
    return pl.pallas_call(
        _mm_body,
        grid=(m // _BLK,),
        in_specs=[
            pl.BlockSpec((_BLK, k), lambda i: (i, 0)),
            pl.BlockSpec((k, n), lambda i: (0, 0)),
        ],
        out_specs=pl.BlockSpec((_BLK, n), lambda i: (i, 0)),
        out_shape=jax.ShapeDtypeStruct((m, n), _f32),
    )(x, w)


def _combine_body(acc_ref, b_ref, w_ref, bw_ref, o_ref):
    nm = acc_ref[0, :, :D_H] + acc_ref[1, :, :D_H]
    dn = jnp.maximum(acc_ref[0, :, D_H] + acc_ref[1, :, D_H], 1e-37)
    h = jnp.maximum(nm / dn[:, None] + b_ref[0], 0.0)
    o_ref[...] = jnp.dot(h, w_ref[...], preferred_element_type=_f32) + bw_ref[0]


def _tc_combine(acc, b, w, bw):
    n_out = w.shape[1]
    return pl.pallas_call(
        _combine_body,
        grid=(_GRID,),
        in_specs=[
            pl.BlockSpec((NC, _BLK, ACC_W), lambda i: (0, i, 0)),
            pl.BlockSpec((1, D_H), lambda i: (0, 0)),
            pl.BlockSpec((D_H, n_out), lambda i: (0, 0)),
            pl.BlockSpec((1, n_out), lambda i: (0, 0)),
        ],
        out_specs=pl.BlockSpec((_BLK, n_out), lambda i: (i, 0)),
        out_shape=jax.ShapeDtypeStruct((N_PAD, n_out), _f32),
    )(acc, b, w, bw)


def _sc_body(xl_hbm, xr_hbm, att_hbm, src_hbm, dst_hbm,
             acc_hbm,
             xl_sh, xr_sh, acc_sh,
             idx_v,
             xls0_v, xrd0_v, xls1_v, xrd1_v, upd0_v, att_v,
             i0a, i0b, i1a, i1b, g0a, g0b, g1a, g1b, s0, s1):
    cid = lax.axis_index("c")
    sid = lax.axis_index("s")
    wid = sid * NC + cid
    row0 = sid * ROWS_PER_TILE

    zero16 = jnp.zeros((16,), _f32)

    def _z(i, _):
        upd0_v[i, pl.ds(0, 16)] = zero16
        upd0_v[i, pl.ds(16, 16)] = zero16
        upd0_v[i, pl.ds(32, 16)] = zero16
        return 0
    lax.fori_loop(0, B, _z, 0)

    def _zcp(i, _):
        pltpu.sync_copy(upd0_v.at[pl.ds(0, 64)],
                        acc_sh.at[pl.ds(row0 + i * 64, 64)])
        return 0
    lax.fori_loop(0, ROWS_PER_TILE // 64, _zcp, 0)

    pltpu.sync_copy(xl_hbm.at[pl.ds(row0, ROWS_PER_TILE)],
                    xl_sh.at[pl.ds(row0, ROWS_PER_TILE)])
    pltpu.sync_copy(xr_hbm.at[pl.ds(row0, ROWS_PER_TILE)],
                    xr_sh.at[pl.ds(row0, ROWS_PER_TILE)])

    pltpu.sync_copy(att_hbm, att_v)
    att0 = att_v[0, pl.ds(0, 16)]
    att1 = att_v[0, pl.ds(16, 16)]

    plsc.subcore_barrier()

    iota16 = lax.broadcasted_iota(jnp.int32, (16,), 0)
    ebase = wid * E_W

    def _compute(xls_v, xrd_v, upd_v):
        def _group(g, _):
            e0 = g * 16
            sums = jnp.zeros((16,), _f32)
            for i in range(16):
                e = e0 + i
                a0 = xls_v[e, pl.ds(0, 16)]
                a1 = xls_v[e, pl.ds(16, 16)]
                s0_ = a0 + xrd_v[e, pl.ds(0, 16)]
                s1_ = a1 + xrd_v[e, pl.ds(16, 16)]
                l0 = jnp.maximum(s0_, 0.2 * s0_)
                l1 = jnp.maximum(s1_, 0.2 * s1_)
                p = l0 * att0 + l1 * att1
                si = jnp.sum(p)
                sums = jnp.where(iota16 == i, si, sums)
            ex = jnp.exp(sums)
            for i in range(16):
                e = e0 + i
                exi = ex[i]
                upd_v[e, pl.ds(0, 16)] = exi * xls_v[e, pl.ds(0, 16)]
                upd_v[e, pl.ds(16, 16)] = exi * xls_v[e, pl.ds(16, 16)]
                upd_v[e, pl.ds(32, 16)] = jnp.where(iota16 == 0, exi, 0.0)
            return 0
        lax.fori_loop(0, B // 16, _group, 0)

    def _pair(t, _):
        base = pl.multiple_of(ebase + t * 2 * B, 8)
        base1 = pl.multiple_of(base + B, 8)
        ld0a = pltpu.async_copy(src_hbm.at[pl.ds(base, B)], idx_v.at[0], i0a)
        ld0b = pltpu.async_copy(dst_hbm.at[pl.ds(base, B)], idx_v.at[1], i0b)
        ld1a = pltpu.async_copy(src_hbm.at[pl.ds(base1, B)], idx_v.at[2], i1a)
        ld1b = pltpu.async_copy(dst_hbm.at[pl.ds(base1, B)], idx_v.at[3], i1b)
        ld0a.wait()
        ld0b.wait()
        cp0a = pltpu.async_copy(xl_sh.at[idx_v.at[0]], xls0_v, g0a)
        cp0b = pltpu.async_copy(xr_sh.at[idx_v.at[1]], xrd0_v, g0b)
        ld1a.wait()
        ld1b.wait()
        cp1a = pltpu.async_copy(xl_sh.at[idx_v.at[2]], xls1_v, g1a)
        cp1b = pltpu.async_copy(xr_sh.at[idx_v.at[3]], xrd1_v, g1b)

        cp0a.wait()
        cp0b.wait()
        _compute(xls0_v, xrd0_v, upd0_v)
        pltpu.sync_copy(upd0_v, acc_sh.at[idx_v.at[1]], add=True)

        cp1a.wait()
        cp1b.wait()
        _compute(xls1_v, xrd1_v, upd0_v)
        pltpu.sync_copy(upd0_v, acc_sh.at[idx_v.at[3]], add=True)
        return 0

    lax.fori_loop(0, PAIRS, _pair, 0)

    plsc.subcore_barrier()

    pltpu.sync_copy(acc_sh.at[pl.ds(row0, ROWS_PER_TILE)],
                    acc_hbm.at[cid, pl.ds(row0, ROWS_PER_TILE)])


@functools.cache
def _make_sc_attention():
  return pl.kernel(
    _sc_body,
    out_type=jax.ShapeDtypeStruct((NC, N_PAD, ACC_W), _f32),
    mesh=plsc.VectorSubcoreMesh(core_axis_name="c", subcore_axis_name="s",
                                num_cores=NC, num_subcores=NS),
    compiler_params=pltpu.CompilerParams(needs_layout_passes=False),
    scratch_types=[
        pltpu.VMEM_SHARED((N_PAD, D_H), _f32),
        pltpu.VMEM_SHARED((N_PAD, D_H), _f32),
        pltpu.VMEM_SHARED((N_PAD, ACC_W), _f32),
        pltpu.VMEM((4, B), jnp.int32),
        pltpu.VMEM((B, D_H), _f32),
        pltpu.VMEM((B, D_H), _f32),
        pltpu.VMEM((B, D_H), _f32),
        pltpu.VMEM((B, D_H), _f32),
        pltpu.VMEM((B, ACC_W), _f32),
        pltpu.VMEM((1, D_H), _f32),
        pltpu.SemaphoreType.DMA,
        pltpu.SemaphoreType.DMA,
        pltpu.SemaphoreType.DMA,
        pltpu.SemaphoreType.DMA,
        pltpu.SemaphoreType.DMA,
        pltpu.SemaphoreType.DMA,
        pltpu.SemaphoreType.DMA,
        pltpu.SemaphoreType.DMA,
        pltpu.SemaphoreType.DMA,
        pltpu.SemaphoreType.DMA,
    ],
  )


def kernel(x, c, adj_t, edge_w, Wl1, Wr1, att1, b1, Wl2, Wr2, att2, b2, Wfc, bfc):
    del c, edge_w
    loop = jnp.arange(N, dtype=jnp.int32)
    npad = E_PAD - E_TOT
    src = jnp.concatenate([adj_t[0], loop,
                           jnp.zeros((npad,), jnp.int32)])
    dst = jnp.concatenate([adj_t[1], loop,
                           jnp.full((npad,), N, jnp.int32)])

    x_pad = jnp.pad(x, ((0, N_PAD - N), (0, 0)))

    xlr1 = _tc_matmul(x_pad, jnp.concatenate([Wl1, Wr1], axis=1))
    xl1 = xlr1[:, :D_H]
    xr1 = xlr1[:, D_H:]
    sc_attention = _make_sc_attention()
    acc1 = sc_attention(xl1, xr1, att1.reshape(1, D_H), src, dst)

    xlr2 = _tc_combine(acc1, b1.reshape(1, D_H),
                       jnp.concatenate([Wl2, Wr2], axis=1),
                       jnp.zeros((1, 2 * D_H), _f32))
    xl2 = xlr2[:, :D_H]
    xr2 = xlr2[:, D_H:]
    acc2 = sc_attention(xl2, xr2, att2.reshape(1, D_H), src, dst)

    out = _tc_combine(acc2, b2.reshape(1, D_H), Wfc,
                      bfc.reshape(1, 1))
    return out[:N]


# speedup vs baseline: 34.8079x; 1.0299x over previous
import functools

import jax
import jax.numpy as jnp
from jax import lax
from jax.experimental import pallas as pl
from jax.experimental.pallas import tpu as pltpu
from jax.experimental.pallas import tpu_sc as plsc

N = 10000
E = 320000
D_IN = 128
D_H = 32

NC = 2
NS = 16
NW = NC * NS

N_PAD = 10240
ROWS_PER_TILE = N_PAD // NS

B = 64
E_TOT = E + N
E_W = 10368
E_PAD = E_W * NW
CHUNKS = E_W // B
PAIRS = CHUNKS // 2

ACC_W = 48

_f32 = jnp.float32


_BLK = 1024
_GRID = N_PAD // _BLK


def _mm_body(x_ref, w_ref, o_ref):
    o_ref[...] = jnp.dot(x_ref[...], w_ref[...], preferred_element_type=_f32)


def _tc_matmul(x, w):
    m, k = x.shape
    n = w.shape[1]
    return pl.pallas_call(
        _mm_body,
        grid=(m // _BLK,),
        in_specs=[
            pl.BlockSpec((_BLK, k), lambda i: (i, 0)),
            pl.BlockSpec((k, n), lambda i: (0, 0)),
        ],
        out_specs=pl.BlockSpec((_BLK, n), lambda i: (i, 0)),
        out_shape=jax.ShapeDtypeStruct((m, n), _f32),
    )(x, w)


def _combine_body(acc_ref, b_ref, w_ref, bw_ref, o_ref):
    nm = acc_ref[0, :, :D_H] + acc_ref[1, :, :D_H]
    dn = jnp.maximum(acc_ref[0, :, D_H] + acc_ref[1, :, D_H], 1e-37)
    h = jnp.maximum(nm / dn[:, None] + b_ref[0], 0.0)
    o_ref[...] = jnp.dot(h, w_ref[...], preferred_element_type=_f32) + bw_ref[0]


def _tc_combine(acc, b, w, bw):
    n_out = w.shape[1]
    return pl.pallas_call(
        _combine_body,
        grid=(_GRID,),
        in_specs=[
            pl.BlockSpec((NC, _BLK, ACC_W), lambda i: (0, i, 0)),
            pl.BlockSpec((1, D_H), lambda i: (0, 0)),
            pl.BlockSpec((D_H, n_out), lambda i: (0, 0)),
            pl.BlockSpec((1, n_out), lambda i: (0, 0)),
        ],
        out_specs=pl.BlockSpec((_BLK, n_out), lambda i: (i, 0)),
        out_shape=jax.ShapeDtypeStruct((N_PAD, n_out), _f32),
    )(acc, b, w, bw)


def _sc_body(xl_hbm, xr_hbm, att_hbm, src_hbm, dst_hbm,
             acc_hbm,
             xl_sh, xr_sh, acc_sh,
             idx_v,
             xls0_v, xrd0_v, xls1_v, xrd1_v, upd0_v, upd1_v, att_v,
             i0a, i0b, i1a, i1b, g0a, g0b, g1a, g1b, s0, s1):
    cid = lax.axis_index("c")
    sid = lax.axis_index("s")
    wid = sid * NC + cid
    row0 = sid * ROWS_PER_TILE

    zero16 = jnp.zeros((16,), _f32)

    def _z(i, _):
        upd0_v[i, pl.ds(0, 16)] = zero16
        upd0_v[i, pl.ds(16, 16)] = zero16
        upd0_v[i, pl.ds(32, 16)] = zero16
        return 0
    lax.fori_loop(0, B, _z, 0)

    def _zcp(i, _):
        pltpu.sync_copy(upd0_v.at[pl.ds(0, 64)],
                        acc_sh.at[pl.ds(row0 + i * 64, 64)])
        return 0
    lax.fori_loop(0, ROWS_PER_TILE // 64, _zcp, 0)

    pltpu.sync_copy(xl_hbm.at[pl.ds(row0, ROWS_PER_TILE)],
                    xl_sh.at[pl.ds(row0, ROWS_PER_TILE)])
    pltpu.sync_copy(xr_hbm.at[pl.ds(row0, ROWS_PER_TILE)],
                    xr_sh.at[pl.ds(row0, ROWS_PER_TILE)])

    pltpu.sync_copy(att_hbm, att_v)
    att0 = att_v[0, pl.ds(0, 16)]
    att1 = att_v[0, pl.ds(16, 16)]

    plsc.subcore_barrier()

    iota16 = lax.broadcasted_iota(jnp.int32, (16,), 0)
    ebase = wid * E_W

    def _compute(xls_v, xrd_v, upd_v):
        def _group(g, _):
            e0 = g * 16
            sums = jnp.zeros((16,), _f32)
            for i in range(16):
                e = e0 + i
                a0 = xls_v[e, pl.ds(0, 16)]
                a1 = xls_v[e, pl.ds(16, 16)]
                s0_ = a0 + xrd_v[e, pl.ds(0, 16)]
                s1_ = a1 + xrd_v[e, pl.ds(16, 16)]
                l0 = jnp.maximum(s0_, 0.2 * s0_)
                l1 = jnp.maximum(s1_, 0.2 * s1_)
                p = l0 * att0 + l1 * att1
                si = jnp.sum(p)
                sums = jnp.where(iota16 == i, si, sums)
            ex = jnp.exp(sums)
            for i in range(16):
                e = e0 + i
                exi = ex[i]
                upd_v[e, pl.ds(0, 16)] = exi * xls_v[e, pl.ds(0, 16)]
                upd_v[e, pl.ds(16, 16)] = exi * xls_v[e, pl.ds(16, 16)]
                upd_v[e, pl.ds(32, 16)] = jnp.where(iota16 == 0, exi, 0.0)
            return 0
        lax.fori_loop(0, B // 16, _group, 0)

    def _pair(t, _):
        base = pl.multiple_of(ebase + t * 2 * B, 8)
        base1 = pl.multiple_of(base + B, 8)
        ld0a = pltpu.async_copy(src_hbm.at[pl.ds(base, B)], idx_v.at[0], i0a)
        ld0b = pltpu.async_copy(dst_hbm.at[pl.ds(base, B)], idx_v.at[1], i0b)
        ld1a = pltpu.async_copy(src_hbm.at[pl.ds(base1, B)], idx_v.at[2], i1a)
        ld1b = pltpu.async_copy(dst_hbm.at[pl.ds(base1, B)], idx_v.at[3], i1b)
        ld0a.wait()
        ld0b.wait()
        cp0a = pltpu.async_copy(xl_sh.at[idx_v.at[0]], xls0_v, g0a)
        cp0b = pltpu.async_copy(xr_sh.at[idx_v.at[1]], xrd0_v, g0b)
        ld1a.wait()
        ld1b.wait()
        cp1a = pltpu.async_copy(xl_sh.at[idx_v.at[2]], xls1_v, g1a)
        cp1b = pltpu.async_copy(xr_sh.at[idx_v.at[3]], xrd1_v, g1b)

        cp0a.wait()
        cp0b.wait()
        _compute(xls0_v, xrd0_v, upd0_v)
        sc0 = pltpu.async_copy(upd0_v, acc_sh.at[idx_v.at[1]], s0, add=True)

        cp1a.wait()
        cp1b.wait()
        _compute(xls1_v, xrd1_v, upd1_v)
        sc0.wait()
        pltpu.sync_copy(upd1_v, acc_sh.at[idx_v.at[3]], add=True)
        return 0

    lax.fori_loop(0, PAIRS, _pair, 0)

    plsc.subcore_barrier()

    pltpu.sync_copy(acc_sh.at[pl.ds(row0, ROWS_PER_TILE)],
                    acc_hbm.at[cid, pl.ds(row0, ROWS_PER_TILE)])


@functools.cache
def _make_sc_attention():
  return pl.kernel(
    _sc_body,
    out_type=jax.ShapeDtypeStruct((NC, N_PAD, ACC_W), _f32),
    mesh=plsc.VectorSubcoreMesh(core_axis_name="c", subcore_axis_name="s",
                                num_cores=NC, num_subcores=NS),
    compiler_params=pltpu.CompilerParams(needs_layout_passes=False),
    scratch_types=[
        pltpu.VMEM_SHARED((N_PAD, D_H), _f32),
        pltpu.VMEM_SHARED((N_PAD, D_H), _f32),
        pltpu.VMEM_SHARED((N_PAD, ACC_W), _f32),
        pltpu.VMEM((4, B), jnp.int32),
        pltpu.VMEM((B, D_H), _f32),
        pltpu.VMEM((B, D_H), _f32),
        pltpu.VMEM((B, D_H), _f32),
        pltpu.VMEM((B, D_H), _f32),
        pltpu.VMEM((B, ACC_W), _f32),
        pltpu.VMEM((B, ACC_W), _f32),
        pltpu.VMEM((1, D_H), _f32),
        pltpu.SemaphoreType.DMA,
        pltpu.SemaphoreType.DMA,
        pltpu.SemaphoreType.DMA,
        pltpu.SemaphoreType.DMA,
        pltpu.SemaphoreType.DMA,
        pltpu.SemaphoreType.DMA,
        pltpu.SemaphoreType.DMA,
        pltpu.SemaphoreType.DMA,
        pltpu.SemaphoreType.DMA,
        pltpu.SemaphoreType.DMA,
    ],
  )


def kernel(x, c, adj_t, edge_w, Wl1, Wr1, att1, b1, Wl2, Wr2, att2, b2, Wfc, bfc):
    del c, edge_w
    loop = jnp.arange(N, dtype=jnp.int32)
    npad = E_PAD - E_TOT
    src = jnp.concatenate([adj_t[0], loop,
                           jnp.zeros((npad,), jnp.int32)])
    dst = jnp.concatenate([adj_t[1], loop,
                           jnp.full((npad,), N, jnp.int32)])

    x_pad = jnp.pad(x, ((0, N_PAD - N), (0, 0)))

    xlr1 = _tc_matmul(x_pad, jnp.concatenate([Wl1, Wr1], axis=1))
    xl1 = xlr1[:, :D_H]
    xr1 = xlr1[:, D_H:]
    sc_attention = _make_sc_attention()
    acc1 = sc_attention(xl1, xr1, att1.reshape(1, D_H), src, dst)

    xlr2 = _tc_combine(acc1, b1.reshape(1, D_H),
                       jnp.concatenate([Wl2, Wr2], axis=1),
                       jnp.zeros((1, 2 * D_H), _f32))
    xl2 = xlr2[:, :D_H]
    xr2 = xlr2[:, D_H:]
    acc2 = sc_attention(xl2, xr2, att2.reshape(1, D_H), src, dst)

    out = _tc_combine(acc2, b2.reshape(1, D_H), Wfc,
                      bfc.reshape(1, 1))
    return out[:N]


# speedup vs baseline: 35.2610x; 1.0130x over previous
import functools

import jax
import jax.numpy as jnp
from jax import lax
from jax.experimental import pallas as pl
from jax.experimental.pallas import tpu as pltpu
from jax.experimental.pallas import tpu_sc as plsc

N = 10000
E = 320000
D_IN = 128
D_H = 32

NC = 2
NS = 16
NW = NC * NS

N_PAD = 10240
ROWS_PER_TILE = N_PAD // NS

B = 72
E_TOT = E + N
E_W = 10368
E_PAD = E_W * NW
CHUNKS = E_W // B
PAIRS = CHUNKS // 2

ACC_W = 48

_f32 = jnp.float32


_BLK = 1024
_GRID = N_PAD // _BLK


def _mm_body(x_ref, w_ref, o_ref):
    o_ref[...] = jnp.dot(x_ref[...], w_ref[...], preferred_element_type=_f32)


def _tc_matmul(x, w):
    m, k = x.shape
    n = w.shape[1]
    return pl.pallas_call(
        _mm_body,
        grid=(m // _BLK,),
        in_specs=[
            pl.BlockSpec((_BLK, k), lambda i: (i, 0)),
            pl.BlockSpec((k, n), lambda i: (0, 0)),
        ],
        out_specs=pl.BlockSpec((_BLK, n), lambda i: (i, 0)),
        out_shape=jax.ShapeDtypeStruct((m, n), _f32),
    )(x, w)


def _combine_body(acc_ref, b_ref, w_ref, bw_ref, o_ref):
    nm = acc_ref[0, :, :D_H] + acc_ref[1, :, :D_H]
    dn = jnp.maximum(acc_ref[0, :, D_H] + acc_ref[1, :, D_H], 1e-37)
    h = jnp.maximum(nm / dn[:, None] + b_ref[0], 0.0)
    o_ref[...] = jnp.dot(h, w_ref[...], preferred_element_type=_f32) + bw_ref[0]


def _tc_combine(acc, b, w, bw):
    n_out = w.shape[1]
    return pl.pallas_call(
        _combine_body,
        grid=(_GRID,),
        in_specs=[
            pl.BlockSpec((NC, _BLK, ACC_W), lambda i: (0, i, 0)),
            pl.BlockSpec((1, D_H), lambda i: (0, 0)),
            pl.BlockSpec((D_H, n_out), lambda i: (0, 0)),
            pl.BlockSpec((1, n_out), lambda i: (0, 0)),
        ],
        out_specs=pl.BlockSpec((_BLK, n_out), lambda i: (i, 0)),
        out_shape=jax.ShapeDtypeStruct((N_PAD, n_out), _f32),
    )(acc, b, w, bw)


def _sc_body(xl_hbm, xr_hbm, att_hbm, sd_hbm,
             acc_hbm,
             xl_sh, xr_sh, acc_sh,
             idx_v,
             xls0_v, xrd0_v, xls1_v, xrd1_v, upd0_v, upd1_v, att_v,
             i0a, i1a, g0a, g0b, g1a, g1b, s0, s1):
    cid = lax.axis_index("c")
    sid = lax.axis_index("s")
    wid = sid * NC + cid
    row0 = sid * ROWS_PER_TILE

    zero16 = jnp.zeros((16,), _f32)

    def _z(i, _):
        upd0_v[i, pl.ds(0, 16)] = zero16
        upd0_v[i, pl.ds(16, 16)] = zero16
        upd0_v[i, pl.ds(32, 16)] = zero16
        return 0
    lax.fori_loop(0, B, _z, 0)

    def _zcp(i, _):
        pltpu.sync_copy(upd0_v.at[pl.ds(0, 64)],
                        acc_sh.at[pl.ds(row0 + i * 64, 64)])
        return 0
    lax.fori_loop(0, ROWS_PER_TILE // 64, _zcp, 0)

    pltpu.sync_copy(xl_hbm.at[pl.ds(row0, ROWS_PER_TILE)],
                    xl_sh.at[pl.ds(row0, ROWS_PER_TILE)])
    pltpu.sync_copy(xr_hbm.at[pl.ds(row0, ROWS_PER_TILE)],
                    xr_sh.at[pl.ds(row0, ROWS_PER_TILE)])

    pltpu.sync_copy(att_hbm, att_v)
    att0 = att_v[0, pl.ds(0, 16)]
    att1 = att_v[0, pl.ds(16, 16)]

    plsc.subcore_barrier()

    iota16 = lax.broadcasted_iota(jnp.int32, (16,), 0)
    cbase = wid * CHUNKS

    def _compute(xls_v, xrd_v, upd_v):
        def _group(g, _):
            e0 = g * 16
            sums = jnp.zeros((16,), _f32)
            for i in range(16):
                e = e0 + i
                a0 = xls_v[e, pl.ds(0, 16)]
                a1 = xls_v[e, pl.ds(16, 16)]
                s0_ = a0 + xrd_v[e, pl.ds(0, 16)]
                s1_ = a1 + xrd_v[e, pl.ds(16, 16)]
                l0 = jnp.maximum(s0_, 0.2 * s0_)
                l1 = jnp.maximum(s1_, 0.2 * s1_)
                p = l0 * att0 + l1 * att1
                si = jnp.sum(p)
                sums = jnp.where(iota16 == i, si, sums)
            ex = jnp.exp(sums)
            for i in range(16):
                e = e0 + i
                exi = ex[i]
                upd_v[e, pl.ds(0, 16)] = exi * xls_v[e, pl.ds(0, 16)]
                upd_v[e, pl.ds(16, 16)] = exi * xls_v[e, pl.ds(16, 16)]
                upd_v[e, pl.ds(32, 16)] = jnp.where(iota16 == 0, exi, 0.0)
            return 0
        lax.fori_loop(0, B // 16, _group, 0)

    def _pair(t, _):
        c0 = cbase + t * 2
        ld0 = pltpu.async_copy(sd_hbm.at[c0], idx_v.at[pl.ds(0, 2)], i0a)
        ld1 = pltpu.async_copy(sd_hbm.at[c0 + 1], idx_v.at[pl.ds(2, 2)], i1a)
        ld0.wait()
        cp0a = pltpu.async_copy(xl_sh.at[idx_v.at[0]], xls0_v, g0a)
        cp0b = pltpu.async_copy(xr_sh.at[idx_v.at[1]], xrd0_v, g0b)
        ld1.wait()
        cp1a = pltpu.async_copy(xl_sh.at[idx_v.at[2]], xls1_v, g1a)
        cp1b = pltpu.async_copy(xr_sh.at[idx_v.at[3]], xrd1_v, g1b)

        cp0a.wait()
        cp0b.wait()
        _compute(xls0_v, xrd0_v, upd0_v)
        sc0 = pltpu.async_copy(upd0_v, acc_sh.at[idx_v.at[1]], s0, add=True)

        cp1a.wait()
        cp1b.wait()
        _compute(xls1_v, xrd1_v, upd1_v)
        sc0.wait()
        pltpu.sync_copy(upd1_v, acc_sh.at[idx_v.at[3]], add=True)
        return 0

    lax.fori_loop(0, PAIRS, _pair, 0)

    plsc.subcore_barrier()

    pltpu.sync_copy(acc_sh.at[pl.ds(row0, ROWS_PER_TILE)],
                    acc_hbm.at[cid, pl.ds(row0, ROWS_PER_TILE)])


@functools.cache
def _make_sc_attention():
  return pl.kernel(
    _sc_body,
    out_type=jax.ShapeDtypeStruct((NC, N_PAD, ACC_W), _f32),
    mesh=plsc.VectorSubcoreMesh(core_axis_name="c", subcore_axis_name="s",
                                num_cores=NC, num_subcores=NS),
    compiler_params=pltpu.CompilerParams(needs_layout_passes=False),
    scratch_types=[
        pltpu.VMEM_SHARED((N_PAD, D_H), _f32),
        pltpu.VMEM_SHARED((N_PAD, D_H), _f32),
        pltpu.VMEM_SHARED((N_PAD, ACC_W), _f32),
        pltpu.VMEM((4, B), jnp.int32),
        pltpu.VMEM((B, D_H), _f32),
        pltpu.VMEM((B, D_H), _f32),
        pltpu.VMEM((B, D_H), _f32),
        pltpu.VMEM((B, D_H), _f32),
        pltpu.VMEM((B, ACC_W), _f32),
        pltpu.VMEM((B, ACC_W), _f32),
        pltpu.VMEM((1, D_H), _f32),
        pltpu.SemaphoreType.DMA,
        pltpu.SemaphoreType.DMA,
        pltpu.SemaphoreType.DMA,
        pltpu.SemaphoreType.DMA,
        pltpu.SemaphoreType.DMA,
        pltpu.SemaphoreType.DMA,
        pltpu.SemaphoreType.DMA,
        pltpu.SemaphoreType.DMA,
    ],
  )


def kernel(x, c, adj_t, edge_w, Wl1, Wr1, att1, b1, Wl2, Wr2, att2, b2, Wfc, bfc):
    del c, edge_w
    loop = jnp.arange(N, dtype=jnp.int32)
    npad = E_PAD - E_TOT
    src = jnp.concatenate([adj_t[0], loop,
                           jnp.zeros((npad,), jnp.int32)])
    dst = jnp.concatenate([adj_t[1], loop,
                           jnp.full((npad,), N, jnp.int32)])
    srcdst = jnp.stack([src.reshape(-1, B), dst.reshape(-1, B)], axis=1)

    x_pad = jnp.pad(x, ((0, N_PAD - N), (0, 0)))

    xlr1 = _tc_matmul(x_pad, jnp.concatenate([Wl1, Wr1], axis=1))
    xl1 = xlr1[:, :D_H]
    xr1 = xlr1[:, D_H:]
    sc_attention = _make_sc_attention()
    acc1 = sc_attention(xl1, xr1, att1.reshape(1, D_H), srcdst)

    xlr2 = _tc_combine(acc1, b1.reshape(1, D_H),
                       jnp.concatenate([Wl2, Wr2], axis=1),
                       jnp.zeros((1, 2 * D_H), _f32))
    xl2 = xlr2[:, :D_H]
    xr2 = xlr2[:, D_H:]
    acc2 = sc_attention(xl2, xr2, att2.reshape(1, D_H), srcdst)

    out = _tc_combine(acc2, b2.reshape(1, D_H), Wfc,
                      bfc.reshape(1, 1))
    return out[:N]


# speedup vs baseline: 36.4017x; 1.0323x over previous
import functools

import jax
import jax.numpy as jnp
from jax import lax
from jax.experimental import pallas as pl
from jax.experimental.pallas import tpu as pltpu
from jax.experimental.pallas import tpu_sc as plsc

N = 10000
E = 320000
D_IN = 128
D_H = 32

NC = 2
NS = 16
NW = NC * NS

N_PAD = 10240
ROWS_PER_TILE = N_PAD // NS
N_TILE = 624
N_TAIL = N - N_TILE * NS

B = 72
E_TOT = E + N
E_W = 10368
E_PAD = E_W * NW
CHUNKS = E_W // B
PAIRS = CHUNKS // 2

ACC_W = 48

_f32 = jnp.float32


_BLK = 1000
_GRID = N // _BLK


def _mm_body(x_ref, w_ref, o1_ref, o2_ref):
    r = jnp.dot(x_ref[...], w_ref[...], preferred_element_type=_f32)
    o1_ref[...] = r[:, :D_H]
    o2_ref[...] = r[:, D_H:]


def _tc_matmul(x, w):
    m, k = x.shape
    return pl.pallas_call(
        _mm_body,
        grid=(m // _BLK,),
        in_specs=[
            pl.BlockSpec((_BLK, k), lambda i: (i, 0)),
            pl.BlockSpec((k, 2 * D_H), lambda i: (0, 0)),
        ],
        out_specs=[pl.BlockSpec((_BLK, D_H), lambda i: (i, 0)),
                   pl.BlockSpec((_BLK, D_H), lambda i: (i, 0))],
        out_shape=[jax.ShapeDtypeStruct((m, D_H), _f32),
                   jax.ShapeDtypeStruct((m, D_H), _f32)],
    )(x, w)


def _agg(acc_ref, b_ref):
    nm = acc_ref[0, :, :D_H] + acc_ref[1, :, :D_H]
    dn = jnp.maximum(acc_ref[0, :, D_H] + acc_ref[1, :, D_H], 1e-37)
    return jnp.maximum(nm / dn[:, None] + b_ref[0], 0.0)


def _combine2_body(acc_ref, b_ref, w_ref, o1_ref, o2_ref):
    r = jnp.dot(_agg(acc_ref, b_ref), w_ref[...], preferred_element_type=_f32)
    o1_ref[...] = r[:, :D_H]
    o2_ref[...] = r[:, D_H:]


def _tc_combine2(acc, b, w):
    return pl.pallas_call(
        _combine2_body,
        grid=(_GRID,),
        in_specs=[
            pl.BlockSpec((NC, _BLK, ACC_W), lambda i: (0, i, 0)),
            pl.BlockSpec((1, D_H), lambda i: (0, 0)),
            pl.BlockSpec((D_H, 2 * D_H), lambda i: (0, 0)),
        ],
        out_specs=[pl.BlockSpec((_BLK, D_H), lambda i: (i, 0)),
                   pl.BlockSpec((_BLK, D_H), lambda i: (i, 0))],
        out_shape=[jax.ShapeDtypeStruct((N, D_H), _f32),
                   jax.ShapeDtypeStruct((N, D_H), _f32)],
    )(acc, b, w)


def _combine_body(acc_ref, b_ref, w_ref, bw_ref, o_ref):
    h = _agg(acc_ref, b_ref)
    o_ref[...] = jnp.dot(h, w_ref[...], preferred_element_type=_f32) + bw_ref[0]


def _tc_combine(acc, b, w, bw):
    n_out = w.shape[1]
    return pl.pallas_call(
        _combine_body,
        grid=(_GRID,),
        in_specs=[
            pl.BlockSpec((NC, _BLK, ACC_W), lambda i: (0, i, 0)),
            pl.BlockSpec((1, D_H), lambda i: (0, 0)),
            pl.BlockSpec((D_H, n_out), lambda i: (0, 0)),
            pl.BlockSpec((1, n_out), lambda i: (0, 0)),
        ],
        out_specs=pl.BlockSpec((_BLK, n_out), lambda i: (i, 0)),
        out_shape=jax.ShapeDtypeStruct((N, n_out), _f32),
    )(acc, b, w, bw)


def _sc_body(xl_hbm, xr_hbm, att_hbm, sd_hbm,
             acc_hbm,
             xl_sh, xr_sh, acc_sh,
             idx_v,
             xls0_v, xrd0_v, xls1_v, xrd1_v, upd0_v, upd1_v, att_v,
             i0a, i1a, g0a, g0b, g1a, g1b, s0, s1):
    cid = lax.axis_index("c")
    sid = lax.axis_index("s")
    wid = sid * NC + cid
    row0 = sid * ROWS_PER_TILE

    zero16 = jnp.zeros((16,), _f32)

    def _z(i, _):
        upd0_v[i, pl.ds(0, 16)] = zero16
        upd0_v[i, pl.ds(16, 16)] = zero16
        upd0_v[i, pl.ds(32, 16)] = zero16
        return 0
    lax.fori_loop(0, B, _z, 0)

    def _zcp(i, _):
        pltpu.sync_copy(upd0_v.at[pl.ds(0, 64)],
                        acc_sh.at[pl.ds(row0 + i * 64, 64)])
        return 0
    lax.fori_loop(0, ROWS_PER_TILE // 64, _zcp, 0)

    row0h = sid * N_TILE
    tail0 = N_TILE * NS
    pltpu.sync_copy(xl_hbm.at[pl.ds(row0h, N_TILE)],
                    xl_sh.at[pl.ds(row0h, N_TILE)])
    pltpu.sync_copy(xr_hbm.at[pl.ds(row0h, N_TILE)],
                    xr_sh.at[pl.ds(row0h, N_TILE)])
    pltpu.sync_copy(xl_hbm.at[pl.ds(tail0, N_TAIL)],
                    xl_sh.at[pl.ds(tail0, N_TAIL)])
    pltpu.sync_copy(xr_hbm.at[pl.ds(tail0, N_TAIL)],
                    xr_sh.at[pl.ds(tail0, N_TAIL)])

    pltpu.sync_copy(att_hbm, att_v)
    att0 = att_v[0, pl.ds(0, 16)]
    att1 = att_v[0, pl.ds(16, 16)]

    plsc.subcore_barrier()

    iota16 = lax.broadcasted_iota(jnp.int32, (16,), 0)
    cbase = wid * CHUNKS

    def _compute(xls_v, xrd_v, upd_v):
        def _group(g, _):
            e0 = g * 16
            sums = jnp.zeros((16,), _f32)
            for i in range(16):
                e = e0 + i
                a0 = xls_v[e, pl.ds(0, 16)]
                a1 = xls_v[e, pl.ds(16, 16)]
                s0_ = a0 + xrd_v[e, pl.ds(0, 16)]
                s1_ = a1 + xrd_v[e, pl.ds(16, 16)]
                l0 = jnp.maximum(s0_, 0.2 * s0_)
                l1 = jnp.maximum(s1_, 0.2 * s1_)
                p = l0 * att0 + l1 * att1
                si = jnp.sum(p)
                sums = jnp.where(iota16 == i, si, sums)
            ex = jnp.exp(sums)
            for i in range(16):
                e = e0 + i
                exi = ex[i]
                upd_v[e, pl.ds(0, 16)] = exi * xls_v[e, pl.ds(0, 16)]
                upd_v[e, pl.ds(16, 16)] = exi * xls_v[e, pl.ds(16, 16)]
                upd_v[e, pl.ds(32, 16)] = jnp.where(iota16 == 0, exi, 0.0)
            return 0
        lax.fori_loop(0, B // 16, _group, 0)

    def _pair(t, _):
        c0 = cbase + t * 2
        ld0 = pltpu.async_copy(sd_hbm.at[c0], idx_v.at[pl.ds(0, 2)], i0a)
        ld1 = pltpu.async_copy(sd_hbm.at[c0 + 1], idx_v.at[pl.ds(2, 2)], i1a)
        ld0.wait()
        cp0a = pltpu.async_copy(xl_sh.at[idx_v.at[0]], xls0_v, g0a)
        cp0b = pltpu.async_copy(xr_sh.at[idx_v.at[1]], xrd0_v, g0b)
        ld1.wait()
        cp1a = pltpu.async_copy(xl_sh.at[idx_v.at[2]], xls1_v, g1a)
        cp1b = pltpu.async_copy(xr_sh.at[idx_v.at[3]], xrd1_v, g1b)

        cp0a.wait()
        cp0b.wait()
        _compute(xls0_v, xrd0_v, upd0_v)
        sc0 = pltpu.async_copy(upd0_v, acc_sh.at[idx_v.at[1]], s0, add=True)

        cp1a.wait()
        cp1b.wait()
        _compute(xls1_v, xrd1_v, upd1_v)
        sc0.wait()
        pltpu.sync_copy(upd1_v, acc_sh.at[idx_v.at[3]], add=True)
        return 0

    lax.fori_loop(0, PAIRS, _pair, 0)

    plsc.subcore_barrier()

    pltpu.sync_copy(acc_sh.at[pl.ds(row0h, N_TILE)],
                    acc_hbm.at[cid, pl.ds(row0h, N_TILE)])
    pltpu.sync_copy(acc_sh.at[pl.ds(tail0, N_TAIL)],
                    acc_hbm.at[cid, pl.ds(tail0, N_TAIL)])


@functools.cache
def _make_sc_attention():
  return pl.kernel(
    _sc_body,
    out_type=jax.ShapeDtypeStruct((NC, N, ACC_W), _f32),
    mesh=plsc.VectorSubcoreMesh(core_axis_name="c", subcore_axis_name="s",
                                num_cores=NC, num_subcores=NS),
    compiler_params=pltpu.CompilerParams(needs_layout_passes=False),
    scratch_types=[
        pltpu.VMEM_SHARED((N_PAD, D_H), _f32),
        pltpu.VMEM_SHARED((N_PAD, D_H), _f32),
        pltpu.VMEM_SHARED((N_PAD, ACC_W), _f32),
        pltpu.VMEM((4, B), jnp.int32),
        pltpu.VMEM((B, D_H), _f32),
        pltpu.VMEM((B, D_H), _f32),
        pltpu.VMEM((B, D_H), _f32),
        pltpu.VMEM((B, D_H), _f32),
        pltpu.VMEM((B, ACC_W), _f32),
        pltpu.VMEM((B, ACC_W), _f32),
        pltpu.VMEM((1, D_H), _f32),
        pltpu.SemaphoreType.DMA,
        pltpu.SemaphoreType.DMA,
        pltpu.SemaphoreType.DMA,
        pltpu.SemaphoreType.DMA,
        pltpu.SemaphoreType.DMA,
        pltpu.SemaphoreType.DMA,
        pltpu.SemaphoreType.DMA,
        pltpu.SemaphoreType.DMA,
    ],
  )


def kernel(x, c, adj_t, edge_w, Wl1, Wr1, att1, b1, Wl2, Wr2, att2, b2, Wfc, bfc):
    del c, edge_w
    loop = jnp.arange(N, dtype=jnp.int32)
    npad = E_PAD - E_TOT
    src = jnp.concatenate([adj_t[0], loop,
                           jnp.zeros((npad,), jnp.int32)])
    dst = jnp.concatenate([adj_t[1], loop,
                           jnp.full((npad,), N, jnp.int32)])
    srcdst = jnp.stack([src.reshape(-1, B), dst.reshape(-1, B)], axis=1)

    xl1, xr1 = _tc_matmul(x, jnp.concatenate([Wl1, Wr1], axis=1))
    sc_attention = _make_sc_attention()
    acc1 = sc_attention(xl1, xr1, att1.reshape(1, D_H), srcdst)

    xl2, xr2 = _tc_combine2(acc1, b1.reshape(1, D_H),
                            jnp.concatenate([Wl2, Wr2], axis=1))
    acc2 = sc_attention(xl2, xr2, att2.reshape(1, D_H), srcdst)

    return _tc_combine(acc2, b2.reshape(1, D_H), Wfc, bfc.reshape(1, 1))
